# baseline (speedup 1.0000x reference)
import functools

import jax
import jax.numpy as jnp
import numpy as np
from jax import lax
from jax.experimental import pallas as pl
from jax.experimental.pallas import tpu as pltpu
from jax.experimental.pallas import tpu_sc as plsc

N_NODES = 10000
N_PAD = 10240
N_EDGES = 320000
CHUNK = 128
N_WORKERS = 32
E_PAD = 327680
IN_CH = 128
MID = 8
HEADS = 8
OUT_CH = 32
NEG_SLOPE = 0.28
EPS = 1e-16


@functools.lru_cache(maxsize=None)
def _make_sc_edge_pass(D, heads, ch):
    assert heads * ch + 16 == D and D % 16 == 0
    msg_groups = (heads * ch) // 16
    rows_per_sub = N_PAD // 16
    chunks_per_worker = E_PAD // (N_WORKERS * CHUNK)
    zcopies = rows_per_sub // CHUNK
    mesh = plsc.VectorSubcoreMesh(core_axis_name="c", subcore_axis_name="s")

    @functools.partial(
        pl.kernel,
        mesh=mesh,
        compiler_params=pltpu.CompilerParams(
            use_tc_tiling_on_sc=False, needs_layout_passes=False),
        out_type=jax.ShapeDtypeStruct((2, N_PAD, D), jnp.float32),
        scratch_types=[
            pltpu.VMEM((CHUNK,), jnp.int32),
            pltpu.VMEM((CHUNK,), jnp.int32),
            pltpu.VMEM((CHUNK, D), jnp.float32),
            pltpu.VMEM((CHUNK, 16), jnp.float32),
            pltpu.VMEM((CHUNK, D), jnp.float32),
            pltpu.VMEM((CHUNK, D), jnp.float32),
            pltpu.VMEM((16,), jnp.float32),
            pltpu.VMEM_SHARED((N_PAD, D), jnp.float32),
            pltpu.SemaphoreType.DMA,
            pltpu.SemaphoreType.DMA,
        ],
    )
    def edge_pass(s_hbm, ad_hbm, src_hbm, dst_hbm, acc_hbm,
                  idx_s, idx_d, rows_v, adrows_v, acc_v, zero_v, exbuf, acc_sh,
                  sem1, sem2):
        c = lax.axis_index("c")
        s = lax.axis_index("s")
        wid = c * 16 + s
        zvec = jnp.zeros((16,), jnp.float32)

        def zrow(k, _):
            i = k // (D // 16)
            j = k % (D // 16)
            zero_v[i, pl.ds(16 * j, 16)] = zvec
            return 0
        lax.fori_loop(0, CHUNK * (D // 16), zrow, 0)
        for t in range(zcopies):
            pltpu.sync_copy(
                zero_v, acc_sh.at[pl.ds(s * rows_per_sub + t * CHUNK, CHUNK)])
        plsc.subcore_barrier()

        def chunk_body(j, _):
            base = (wid * chunks_per_worker + j) * CHUNK
            pltpu.sync_copy(src_hbm.at[pl.ds(base, CHUNK)], idx_s)
            pltpu.sync_copy(dst_hbm.at[pl.ds(base, CHUNK)], idx_d)
            cp1 = pltpu.async_copy(s_hbm.at[idx_s], rows_v, sem1)
            cp2 = pltpu.async_copy(ad_hbm.at[idx_d], adrows_v, sem2)
            cp1.wait()
            cp2.wait()

            def edge_body(i, _):
                t_ = rows_v[i, pl.ds(heads * ch, 16)] + adrows_v[i, :]
                t_ = jnp.where(t_ >= 0, t_, NEG_SLOPE * t_)
                ex = jnp.exp(t_)
                acc_v[i, pl.ds(heads * ch, 16)] = ex
                if heads == 1:
                    for g in range(msg_groups):
                        acc_v[i, pl.ds(16 * g, 16)] = (
                            rows_v[i, pl.ds(16 * g, 16)] * ex)
                    return 0
                lane = lax.broadcasted_iota(jnp.int32, (16,), 0)
                row_idx = jnp.full((16,), i, jnp.int32)
                for g in range(msg_groups):
                    base = (16 * g) // ch
                    thr = min(16, ch - (16 * g) % ch)
                    hidx = base + (lane >= thr).astype(jnp.int32)
                    exg = plsc.load_gather(acc_v, [row_idx,
                                                   heads * ch + hidx])
                    acc_v[i, pl.ds(16 * g, 16)] = (
                        rows_v[i, pl.ds(16 * g, 16)] * exg)
                return 0
            lax.fori_loop(0, CHUNK, edge_body, 0, unroll=4)
            pltpu.sync_copy(acc_v, acc_sh.at[idx_d], add=True)
            return 0
        lax.fori_loop(0, chunks_per_worker, chunk_body, 0)
        plsc.subcore_barrier()
        pltpu.sync_copy(acc_sh.at[pl.ds(s * rows_per_sub, rows_per_sub)],
                        acc_hbm.at[c, pl.ds(s * rows_per_sub, rows_per_sub)])

    return edge_pass


_ROWS = 1280


def _k1_body(x_ref, w_ref, as_ref, ad_ref, o_ref):
    h = jnp.dot(x_ref[...], w_ref[...], preferred_element_type=jnp.float32)
    ps = h * as_ref[...]
    pd = h * ad_ref[...]
    o_ref[:, :HEADS * MID] = h
    z8 = jnp.zeros((_ROWS, HEADS), jnp.float32)
    o_ref[:, 72:80] = z8
    o_ref[:, 88:96] = z8
    for hd in range(HEADS):
        lo, hi = hd * MID, (hd + 1) * MID
        o_ref[:, 64 + hd:65 + hd] = jnp.sum(ps[:, lo:hi], axis=1,
                                            keepdims=True)
        o_ref[:, 80 + hd:81 + hd] = jnp.sum(pd[:, lo:hi], axis=1,
                                            keepdims=True)


def _k1(xp, w1, a_s, a_d):
    return pl.pallas_call(
        _k1_body,
        grid=(N_PAD // _ROWS,),
        in_specs=[pl.BlockSpec((_ROWS, IN_CH), lambda i: (i, 0)),
                  pl.BlockSpec((IN_CH, 64), lambda i: (0, 0)),
                  pl.BlockSpec((1, 64), lambda i: (0, 0)),
                  pl.BlockSpec((1, 64), lambda i: (0, 0))],
        out_specs=pl.BlockSpec((_ROWS, 96), lambda i: (i, 0)),
        out_shape=jax.ShapeDtypeStruct((N_PAD, 96), jnp.float32),
    )(xp, w1, a_s, a_d)


def _k2_body(aa_ref, ab_ref, bias_ref, w_ref, as_ref, ad_ref, o_ref):
    a = aa_ref[...] + ab_ref[...]
    msg = a[:, : HEADS * MID]
    den = a[:, HEADS * MID: HEADS * MID + HEADS]
    segs = []
    for hd in range(HEADS):
        segs.append(msg[:, hd * MID:(hd + 1) * MID]
                    / (den[:, hd:hd + 1] + EPS))
    x2 = jnp.concatenate(segs, axis=1) + bias_ref[...]
    x2 = jnp.where(x2 > 0, x2, jnp.exp(jnp.minimum(x2, 0.0)) - 1.0)
    h2 = jnp.dot(x2, w_ref[...], preferred_element_type=jnp.float32)
    o_ref[:, :OUT_CH] = h2
    as2 = jnp.sum(h2 * as_ref[...], axis=1, keepdims=True)
    ad2 = jnp.sum(h2 * ad_ref[...], axis=1, keepdims=True)
    for k in range(16):
        o_ref[:, 32 + k:33 + k] = as2
        o_ref[:, 48 + k:49 + k] = ad2


def _k2(a1a, a1b, b1, w2, a_s2, a_d2):
    return pl.pallas_call(
        _k2_body,
        grid=(N_PAD // _ROWS,),
        in_specs=[pl.BlockSpec((_ROWS, 80), lambda i: (i, 0)),
                  pl.BlockSpec((_ROWS, 80), lambda i: (i, 0)),
                  pl.BlockSpec((1, 64), lambda i: (0, 0)),
                  pl.BlockSpec((64, OUT_CH), lambda i: (0, 0)),
                  pl.BlockSpec((1, OUT_CH), lambda i: (0, 0)),
                  pl.BlockSpec((1, OUT_CH), lambda i: (0, 0))],
        out_specs=pl.BlockSpec((_ROWS, 64), lambda i: (i, 0)),
        out_shape=jax.ShapeDtypeStruct((N_PAD, 64), jnp.float32),
    )(a1a, a1b, b1, w2, a_s2, a_d2)


def _k3_body(aa_ref, ab_ref, bias_ref, o_ref):
    a = aa_ref[...] + ab_ref[...]
    o = a[:, :OUT_CH] / (a[:, OUT_CH:OUT_CH + 1] + EPS) + bias_ref[...]
    o = o - jnp.max(o, axis=1, keepdims=True)
    o_ref[...] = o - jnp.log(jnp.sum(jnp.exp(o), axis=1, keepdims=True))


def _k3(a2a, a2b, b2):
    return pl.pallas_call(
        _k3_body,
        grid=(N_PAD // _ROWS,),
        in_specs=[pl.BlockSpec((_ROWS, 48), lambda i: (i, 0)),
                  pl.BlockSpec((_ROWS, 48), lambda i: (i, 0)),
                  pl.BlockSpec((1, OUT_CH), lambda i: (0, 0))],
        out_specs=pl.BlockSpec((_ROWS, OUT_CH), lambda i: (i, 0)),
        out_shape=jax.ShapeDtypeStruct((N_PAD, OUT_CH), jnp.float32),
    )(a2a, a2b, b2)


def kernel(node_feature, adj_list, W1, a_src1, a_dst1, b1,
           W2, a_src2, a_dst2, b2):
    xp = jnp.pad(node_feature, ((0, N_PAD - N_NODES), (0, 0)))
    pad_idx = jnp.full((E_PAD - N_EDGES,), N_NODES, jnp.int32)
    src = jnp.concatenate([adj_list[0].astype(jnp.int32), pad_idx])
    dst = jnp.concatenate([adj_list[1].astype(jnp.int32), pad_idx])

    t1 = _k1(xp, W1, a_src1.reshape(1, HEADS * MID),
             a_dst1.reshape(1, HEADS * MID))
    s1 = t1[:, :80]
    ad1 = t1[:, 80:96]
    acc1 = _make_sc_edge_pass(80, HEADS, MID)(s1, ad1, src, dst)

    t2 = _k2(acc1[0], acc1[1], b1.reshape(1, 64), W2,
             a_src2.reshape(1, OUT_CH), a_dst2.reshape(1, OUT_CH))
    s2 = t2[:, :48]
    ad2 = t2[:, 48:64]
    acc2 = _make_sc_edge_pass(48, 1, OUT_CH)(s2, ad2, src, dst)

    out = _k3(acc2[0], acc2[1], b2.reshape(1, OUT_CH))
    return out[:N_NODES]


# speedup vs baseline: 1.0599x; 1.0599x over previous
import functools

import jax
import jax.numpy as jnp
import numpy as np
from jax import lax
from jax.experimental import pallas as pl
from jax.experimental.pallas import tpu as pltpu
from jax.experimental.pallas import tpu_sc as plsc

N_NODES = 10000
N_PAD = 10240
N_EDGES = 320000
CHUNK = 128
N_WORKERS = 32
E_PAD = 327680
IN_CH = 128
MID = 8
HEADS = 8
OUT_CH = 32
NEG_SLOPE = 0.28
EPS = 1e-16


@functools.lru_cache(maxsize=None)
def _make_sc_edge_pass(D, heads, ch):
    assert heads * ch + 16 == D and D % 16 == 0
    msg_groups = (heads * ch) // 16
    rows_per_sub = N_PAD // 16
    chunks_per_worker = E_PAD // (N_WORKERS * CHUNK)
    zcopies = rows_per_sub // CHUNK
    mesh = plsc.VectorSubcoreMesh(core_axis_name="c", subcore_axis_name="s")

    @functools.partial(
        pl.kernel,
        mesh=mesh,
        compiler_params=pltpu.CompilerParams(
            use_tc_tiling_on_sc=False, needs_layout_passes=False),
        out_type=jax.ShapeDtypeStruct((2, N_PAD, D), jnp.float32),
        scratch_types=[
            pltpu.VMEM((CHUNK,), jnp.int32),
            pltpu.VMEM((CHUNK,), jnp.int32),
            pltpu.VMEM((CHUNK, D), jnp.float32),
            pltpu.VMEM((CHUNK, 16), jnp.float32),
            pltpu.VMEM((CHUNK, D), jnp.float32),
            pltpu.VMEM((CHUNK, D), jnp.float32),
            pltpu.VMEM((16,), jnp.float32),
            pltpu.VMEM_SHARED((N_PAD, D), jnp.float32),
            pltpu.SemaphoreType.DMA,
            pltpu.SemaphoreType.DMA,
        ],
    )
    def edge_pass(s_hbm, ad_hbm, src_hbm, dst_hbm, acc_hbm,
                  idx_s, idx_d, rows_v, adrows_v, acc_v, zero_v, exbuf, acc_sh,
                  sem1, sem2):
        c = lax.axis_index("c")
        s = lax.axis_index("s")
        wid = c * 16 + s
        zvec = jnp.zeros((16,), jnp.float32)

        def zrow(k, _):
            i = k // (D // 16)
            j = k % (D // 16)
            zero_v[i, pl.ds(16 * j, 16)] = zvec
            return 0
        lax.fori_loop(0, CHUNK * (D // 16), zrow, 0)
        for t in range(zcopies):
            pltpu.sync_copy(
                zero_v, acc_sh.at[pl.ds(s * rows_per_sub + t * CHUNK, CHUNK)])
        plsc.subcore_barrier()

        def chunk_body(j, _):
            base = (wid * chunks_per_worker + j) * CHUNK
            pltpu.sync_copy(src_hbm.at[pl.ds(base, CHUNK)], idx_s)
            pltpu.sync_copy(dst_hbm.at[pl.ds(base, CHUNK)], idx_d)
            cp1 = pltpu.async_copy(s_hbm.at[idx_s], rows_v, sem1)
            cp2 = pltpu.async_copy(ad_hbm.at[idx_d], adrows_v, sem2)
            cp1.wait()
            cp2.wait()

            def edge_body(i, _):
                t_ = rows_v[i, pl.ds(heads * ch, 16)] + adrows_v[i, :]
                t_ = jnp.where(t_ >= 0, t_, NEG_SLOPE * t_)
                ex = jnp.exp(t_)
                acc_v[i, pl.ds(heads * ch, 16)] = ex
                if heads == 1:
                    for g in range(msg_groups):
                        acc_v[i, pl.ds(16 * g, 16)] = (
                            rows_v[i, pl.ds(16 * g, 16)] * ex)
                    return 0
                lane = lax.broadcasted_iota(jnp.int32, (16,), 0)
                row_idx = jnp.full((16,), i, jnp.int32)
                for g in range(msg_groups):
                    base = (16 * g) // ch
                    thr = min(16, ch - (16 * g) % ch)
                    hidx = base + (lane >= thr).astype(jnp.int32)
                    exg = plsc.load_gather(acc_v, [row_idx,
                                                   heads * ch + hidx])
                    acc_v[i, pl.ds(16 * g, 16)] = (
                        rows_v[i, pl.ds(16 * g, 16)] * exg)
                return 0
            lax.fori_loop(0, CHUNK, edge_body, 0)
            pltpu.sync_copy(acc_v, acc_sh.at[idx_d], add=True)
            return 0
        lax.fori_loop(0, chunks_per_worker, chunk_body, 0)
        plsc.subcore_barrier()
        pltpu.sync_copy(acc_sh.at[pl.ds(s * rows_per_sub, rows_per_sub)],
                        acc_hbm.at[c, pl.ds(s * rows_per_sub, rows_per_sub)])

    return edge_pass


_ROWS = 1280


def _k1_body(x_ref, w_ref, as_ref, ad_ref, o_ref):
    h = jnp.dot(x_ref[...], w_ref[...], preferred_element_type=jnp.float32)
    ps = h * as_ref[...]
    pd = h * ad_ref[...]
    o_ref[:, :HEADS * MID] = h
    z8 = jnp.zeros((_ROWS, HEADS), jnp.float32)
    o_ref[:, 72:80] = z8
    o_ref[:, 88:96] = z8
    for hd in range(HEADS):
        lo, hi = hd * MID, (hd + 1) * MID
        o_ref[:, 64 + hd:65 + hd] = jnp.sum(ps[:, lo:hi], axis=1,
                                            keepdims=True)
        o_ref[:, 80 + hd:81 + hd] = jnp.sum(pd[:, lo:hi], axis=1,
                                            keepdims=True)


def _k1(xp, w1, a_s, a_d):
    return pl.pallas_call(
        _k1_body,
        grid=(N_PAD // _ROWS,),
        in_specs=[pl.BlockSpec((_ROWS, IN_CH), lambda i: (i, 0)),
                  pl.BlockSpec((IN_CH, 64), lambda i: (0, 0)),
                  pl.BlockSpec((1, 64), lambda i: (0, 0)),
                  pl.BlockSpec((1, 64), lambda i: (0, 0))],
        out_specs=pl.BlockSpec((_ROWS, 96), lambda i: (i, 0)),
        out_shape=jax.ShapeDtypeStruct((N_PAD, 96), jnp.float32),
    )(xp, w1, a_s, a_d)


def _k2_body(aa_ref, ab_ref, bias_ref, w_ref, as_ref, ad_ref, o_ref):
    a = aa_ref[...] + ab_ref[...]
    msg = a[:, : HEADS * MID]
    den = a[:, HEADS * MID: HEADS * MID + HEADS]
    segs = []
    for hd in range(HEADS):
        segs.append(msg[:, hd * MID:(hd + 1) * MID]
                    / (den[:, hd:hd + 1] + EPS))
    x2 = jnp.concatenate(segs, axis=1) + bias_ref[...]
    x2 = jnp.where(x2 > 0, x2, jnp.exp(jnp.minimum(x2, 0.0)) - 1.0)
    h2 = jnp.dot(x2, w_ref[...], preferred_element_type=jnp.float32)
    o_ref[:, :OUT_CH] = h2
    as2 = jnp.sum(h2 * as_ref[...], axis=1, keepdims=True)
    ad2 = jnp.sum(h2 * ad_ref[...], axis=1, keepdims=True)
    for k in range(16):
        o_ref[:, 32 + k:33 + k] = as2
        o_ref[:, 48 + k:49 + k] = ad2


def _k2(a1a, a1b, b1, w2, a_s2, a_d2):
    return pl.pallas_call(
        _k2_body,
        grid=(N_PAD // _ROWS,),
        in_specs=[pl.BlockSpec((_ROWS, 80), lambda i: (i, 0)),
                  pl.BlockSpec((_ROWS, 80), lambda i: (i, 0)),
                  pl.BlockSpec((1, 64), lambda i: (0, 0)),
                  pl.BlockSpec((64, OUT_CH), lambda i: (0, 0)),
                  pl.BlockSpec((1, OUT_CH), lambda i: (0, 0)),
                  pl.BlockSpec((1, OUT_CH), lambda i: (0, 0))],
        out_specs=pl.BlockSpec((_ROWS, 64), lambda i: (i, 0)),
        out_shape=jax.ShapeDtypeStruct((N_PAD, 64), jnp.float32),
    )(a1a, a1b, b1, w2, a_s2, a_d2)


def _k3_body(aa_ref, ab_ref, bias_ref, o_ref):
    a = aa_ref[...] + ab_ref[...]
    o = a[:, :OUT_CH] / (a[:, OUT_CH:OUT_CH + 1] + EPS) + bias_ref[...]
    o = o - jnp.max(o, axis=1, keepdims=True)
    o_ref[...] = o - jnp.log(jnp.sum(jnp.exp(o), axis=1, keepdims=True))


def _k3(a2a, a2b, b2):
    return pl.pallas_call(
        _k3_body,
        grid=(N_PAD // _ROWS,),
        in_specs=[pl.BlockSpec((_ROWS, 48), lambda i: (i, 0)),
                  pl.BlockSpec((_ROWS, 48), lambda i: (i, 0)),
                  pl.BlockSpec((1, OUT_CH), lambda i: (0, 0))],
        out_specs=pl.BlockSpec((_ROWS, OUT_CH), lambda i: (i, 0)),
        out_shape=jax.ShapeDtypeStruct((N_PAD, OUT_CH), jnp.float32),
    )(a2a, a2b, b2)


def kernel(node_feature, adj_list, W1, a_src1, a_dst1, b1,
           W2, a_src2, a_dst2, b2):
    xp = jnp.pad(node_feature, ((0, N_PAD - N_NODES), (0, 0)))
    pad_idx = jnp.full((E_PAD - N_EDGES,), N_NODES, jnp.int32)
    src = jnp.concatenate([adj_list[0].astype(jnp.int32), pad_idx])
    dst = jnp.concatenate([adj_list[1].astype(jnp.int32), pad_idx])

    t1 = _k1(xp, W1, a_src1.reshape(1, HEADS * MID),
             a_dst1.reshape(1, HEADS * MID))
    s1 = t1[:, :80]
    ad1 = t1[:, 80:96]
    acc1 = _make_sc_edge_pass(80, HEADS, MID)(s1, ad1, src, dst)

    t2 = _k2(acc1[0], acc1[1], b1.reshape(1, 64), W2,
             a_src2.reshape(1, OUT_CH), a_dst2.reshape(1, OUT_CH))
    s2 = t2[:, :48]
    ad2 = t2[:, 48:64]
    acc2 = _make_sc_edge_pass(48, 1, OUT_CH)(s2, ad2, src, dst)

    out = _k3(acc2[0], acc2[1], b2.reshape(1, OUT_CH))
    return out[:N_NODES]
